# baseline (speedup 1.0000x reference)
import jax
import jax.numpy as jnp
import numpy as np
from jax import lax
from jax.experimental import pallas as pl
from jax.experimental.pallas import tpu as pltpu
from jax.experimental.pallas import tpu_sc as plsc

N_NODES = 10000
DIM = 128
N_EDGES = 320000

NC = 2
NS = 16
LANES = 16

CHUNK = 128
N_CHUNKS_PAD = 2560
N_EDGES_PAD = N_CHUNKS_PAD * CHUNK
CHUNKS_PER_TILE = N_CHUNKS_PAD // NS
VREGS_PER_ROW = DIM // LANES

RCHUNK = 80
N_RCHUNKS = N_NODES // RCHUNK
RCHUNKS_PER_TILE = -(-N_RCHUNKS // NS)


G = 32
N_GROUPS = CHUNKS_PER_TILE // G


def _sc_body(src_hbm, dst_hbm, attr_hbm, x_hbm, out_hbm,
             acc, idx_s, idx_d, attr_a, rows0, rows1, msg,
             semi, sem0, sem1):
    c = lax.axis_index("c")
    t = lax.axis_index("s")

    def _issue_idx(g):
        base = t * CHUNKS_PER_TILE + g * G
        pltpu.async_copy(src_hbm.at[c, pl.ds(base, G)], idx_s, semi)
        pltpu.async_copy(dst_hbm.at[c, pl.ds(base, G)], idx_d, semi)
        pltpu.async_copy(attr_hbm.at[c, pl.ds(base, G)], attr_a, semi)

    def _wait_idx(g):
        base = t * CHUNKS_PER_TILE + g * G
        pltpu.make_async_copy(src_hbm.at[c, pl.ds(base, G)], idx_s,
                              semi).wait()
        pltpu.make_async_copy(dst_hbm.at[c, pl.ds(base, G)], idx_d,
                              semi).wait()
        pltpu.make_async_copy(attr_hbm.at[c, pl.ds(base, G)], attr_a,
                              semi).wait()

    _issue_idx(0)

    @pl.loop(0, CHUNK)
    def _zero_rows(r):
        for k in range(VREGS_PER_ROW):
            msg[r, pl.ds(k * LANES, LANES)] = jnp.zeros((LANES,), jnp.float32)

    @pl.loop(0, RCHUNKS_PER_TILE)
    def _zero_acc(i):
        rc = i * NS + t

        @pl.when(rc < N_RCHUNKS)
        def _():
            pltpu.sync_copy(msg.at[pl.ds(0, RCHUNK)],
                            acc.at[pl.ds(rc * RCHUNK, RCHUNK)])

    plsc.subcore_barrier()

    def _process(j, cur, nxt, sem_cur, sem_nxt):
        @pl.when(j + 1 < G)
        def _():
            pltpu.async_copy(x_hbm.at[idx_s.at[j + 1]], nxt, sem_nxt)

        pltpu.make_async_copy(x_hbm.at[idx_s.at[j]], cur, sem_cur).wait()

        @plsc.parallel_loop(0, CHUNK // LANES, unroll=2)
        def _scale(gg):
            a16 = attr_a[j, pl.ds(gg * LANES, LANES)]
            for jj in range(LANES):
                av = jnp.full((LANES,), a16[jj], jnp.float32)
                e = gg * LANES + jj
                for k in range(VREGS_PER_ROW // 2):
                    w = cur[e, pl.ds(k * LANES, LANES)]
                    lo = lax.bitcast_convert_type(w << 16, jnp.float32)
                    hi = lax.bitcast_convert_type(w & jnp.int32(-65536),
                                                  jnp.float32)
                    sl_lo = pl.ds(k * 2 * LANES, LANES)
                    sl_hi = pl.ds(k * 2 * LANES + LANES, LANES)
                    msg[e, sl_lo] = lo * av
                    msg[e, sl_hi] = hi * av

        pltpu.sync_copy(msg, acc.at[idx_d.at[j]], add=True)

    @pl.loop(0, N_GROUPS)
    def _group_loop(g):
        @pl.when(g > 0)
        def _():
            _issue_idx(g)

        _wait_idx(g)
        pltpu.async_copy(x_hbm.at[idx_s.at[0]], rows0, sem0)

        @pl.loop(0, G)
        def _chunk_loop(j):
            @pl.when(j % 2 == 0)
            def _():
                _process(j, rows0, rows1, sem0, sem1)

            @pl.when(j % 2 == 1)
            def _():
                _process(j, rows1, rows0, sem1, sem0)

    plsc.subcore_barrier()

    @pl.loop(0, RCHUNKS_PER_TILE)
    def _drain(i):
        rc = i * NS + t

        @pl.when(rc < N_RCHUNKS)
        def _():
            pltpu.sync_copy(acc.at[pl.ds(rc * RCHUNK, RCHUNK)],
                            msg.at[pl.ds(0, RCHUNK)])
            pltpu.sync_copy(msg.at[pl.ds(0, RCHUNK)],
                            out_hbm.at[c, pl.ds(rc * RCHUNK, RCHUNK)])


_SC_AGG_CACHE = []


def _sc_agg_fn():
    if not _SC_AGG_CACHE:
        _SC_AGG_CACHE.append(pl.kernel(
            _sc_body,
            out_type=jax.ShapeDtypeStruct((NC, N_NODES, DIM), jnp.float32),
            mesh=plsc.VectorSubcoreMesh(core_axis_name="c",
                                        subcore_axis_name="s",
                                        num_cores=NC, num_subcores=NS),
            compiler_params=pltpu.CompilerParams(use_tc_tiling_on_sc=False),
            scratch_types=[
                pltpu.VMEM_SHARED((N_NODES, DIM), jnp.float32),
                pltpu.VMEM((G, CHUNK), jnp.int32),
                pltpu.VMEM((G, CHUNK), jnp.int32),
                pltpu.VMEM((G, CHUNK), jnp.float32),
                pltpu.VMEM((CHUNK, DIM // 2), jnp.int32),
                pltpu.VMEM((CHUNK, DIM // 2), jnp.int32),
                pltpu.VMEM((CHUNK, DIM), jnp.float32),
                pltpu.SemaphoreType.DMA,
                pltpu.SemaphoreType.DMA,
                pltpu.SemaphoreType.DMA,
            ],
        ))
    return _SC_AGG_CACHE[0]


BM = 1000


def _mm_body(x_ref, a1_ref, a2_ref, wln_ref, w1_ref, w2_ref,
             o0_ref, o1_ref, o2_ref):
    o0_ref[...] = jnp.dot(x_ref[...], wln_ref[...],
                          preferred_element_type=jnp.float32)
    o1_ref[...] = jnp.dot(a1_ref[0], w1_ref[...],
                          preferred_element_type=jnp.float32)
    o2_ref[...] = jnp.dot(a2_ref[0], w2_ref[...],
                          preferred_element_type=jnp.float32)


_row_spec = pl.BlockSpec((BM, DIM), lambda i: (i, 0))
_w_spec = pl.BlockSpec((DIM, DIM), lambda i: (0, 0))

_tc_matmul = pl.pallas_call(
    _mm_body,
    grid=(N_NODES // BM,),
    in_specs=[_row_spec,
              pl.BlockSpec((1, BM, DIM), lambda i: (0, i, 0)),
              pl.BlockSpec((1, BM, DIM), lambda i: (1, i, 0)),
              _w_spec, _w_spec, _w_spec],
    out_specs=[_row_spec, _row_spec, _row_spec],
    out_shape=[jax.ShapeDtypeStruct((N_NODES, DIM), jnp.float32)] * 3,
)


_PERM = np.asarray(
    [32 * k + (j % 2) * 16 + j // 2 for k in range(DIM // 32)
     for j in range(32)], dtype=np.int32)


def kernel(x, edge_index, edge_attr, edge_index2, edge_attr2, W_ln, W1, W2):
    ei1 = edge_index.astype(jnp.int32)
    ei2 = edge_index2.astype(jnp.int32)
    x_bf = x.astype(jnp.bfloat16)[:, _PERM]
    x_pk = jax.lax.bitcast_convert_type(
        x_bf.reshape(N_NODES, DIM // 2, 2), jnp.int32)
    npad = N_EDGES_PAD - N_EDGES
    pad_idx = (jnp.arange(npad, dtype=jnp.int32) * 8) % N_NODES
    pad_attr = jnp.zeros((npad,), jnp.float32)

    def _pad(a, p):
        return jnp.concatenate([a, p])

    src = jnp.stack([_pad(ei1[0], pad_idx), _pad(ei2[0], pad_idx)])
    dst = jnp.stack([_pad(ei1[1], pad_idx), _pad(ei2[1], pad_idx)])
    attr = jnp.stack([_pad(edge_attr, pad_attr), _pad(edge_attr2, pad_attr)])
    src = src.reshape(NC, N_CHUNKS_PAD, CHUNK)
    dst = dst.reshape(NC, N_CHUNKS_PAD, CHUNK)
    attr = attr.reshape(NC, N_CHUNKS_PAD, CHUNK)
    agg = _sc_agg_fn()(src, dst, attr, x_pk)
    x0, x1, x2 = _tc_matmul(x, agg, agg, W_ln, W1, W2)
    return (x0, x1, x2)


# speedup vs baseline: 1.4718x; 1.4718x over previous
import jax
import jax.numpy as jnp
from jax import lax
from jax.experimental import pallas as pl
from jax.experimental.pallas import tpu as pltpu
from jax.experimental.pallas import tpu_sc as plsc

N_NODES = 10000
DIM = 128
N_EDGES = 320000

NC = 2
NS = 16
LANES = 16

CHUNK = 128
N_CHUNKS_PAD = 2560
N_EDGES_PAD = N_CHUNKS_PAD * CHUNK
CHUNKS_PER_TILE = N_CHUNKS_PAD // NS
VREGS_PER_ROW = DIM // LANES

RCHUNK = 80
N_RCHUNKS = N_NODES // RCHUNK
RCHUNKS_PER_TILE = -(-N_RCHUNKS // NS)


G = 32
N_GROUPS = CHUNKS_PER_TILE // G


def _sc_body(src_hbm, dst_hbm, attr_hbm, x_hbm, out_hbm,
             acc, idx_s, idx_d, attr_a, rows0, rows1,
             semi, sem0, sem1):
    c = lax.axis_index("c")
    t = lax.axis_index("s")

    def _issue_idx(g):
        base = t * CHUNKS_PER_TILE + g * G
        pltpu.async_copy(src_hbm.at[c, pl.ds(base, G)], idx_s, semi)
        pltpu.async_copy(dst_hbm.at[c, pl.ds(base, G)], idx_d, semi)
        pltpu.async_copy(attr_hbm.at[c, pl.ds(base, G)], attr_a, semi)

    def _wait_idx(g):
        base = t * CHUNKS_PER_TILE + g * G
        pltpu.make_async_copy(src_hbm.at[c, pl.ds(base, G)], idx_s,
                              semi).wait()
        pltpu.make_async_copy(dst_hbm.at[c, pl.ds(base, G)], idx_d,
                              semi).wait()
        pltpu.make_async_copy(attr_hbm.at[c, pl.ds(base, G)], attr_a,
                              semi).wait()

    _issue_idx(0)

    @pl.loop(0, CHUNK)
    def _zero_rows(r):
        for k in range(VREGS_PER_ROW):
            rows0[r, pl.ds(k * LANES, LANES)] = jnp.zeros((LANES,), jnp.float32)

    @pl.loop(0, RCHUNKS_PER_TILE)
    def _zero_acc(i):
        rc = i * NS + t

        @pl.when(rc < N_RCHUNKS)
        def _():
            pltpu.sync_copy(rows0.at[pl.ds(0, RCHUNK)],
                            acc.at[pl.ds(rc * RCHUNK, RCHUNK)])

    plsc.subcore_barrier()

    def _process(j, cur, nxt, sem_cur, sem_nxt):
        @pl.when(j + 1 < G)
        def _():
            pltpu.async_copy(x_hbm.at[idx_s.at[j + 1]], nxt, sem_nxt)

        pltpu.make_async_copy(x_hbm.at[idx_s.at[j]], cur, sem_cur).wait()

        @plsc.parallel_loop(0, CHUNK // LANES, unroll=2)
        def _scale(gg):
            a16 = attr_a[j, pl.ds(gg * LANES, LANES)]
            for jj in range(LANES):
                av = jnp.full((LANES,), a16[jj], jnp.float32)
                e = gg * LANES + jj
                for k in range(VREGS_PER_ROW):
                    sl = pl.ds(k * LANES, LANES)
                    cur[e, sl] = cur[e, sl] * av

        pltpu.sync_copy(cur, acc.at[idx_d.at[j]], add=True)

    @pl.loop(0, N_GROUPS)
    def _group_loop(g):
        @pl.when(g > 0)
        def _():
            _issue_idx(g)

        _wait_idx(g)
        pltpu.async_copy(x_hbm.at[idx_s.at[0]], rows0, sem0)

        @pl.loop(0, G)
        def _chunk_loop(j):
            @pl.when(j % 2 == 0)
            def _():
                _process(j, rows0, rows1, sem0, sem1)

            @pl.when(j % 2 == 1)
            def _():
                _process(j, rows1, rows0, sem1, sem0)

    plsc.subcore_barrier()

    @pl.loop(0, RCHUNKS_PER_TILE)
    def _drain(i):
        rc = i * NS + t

        @pl.when(rc < N_RCHUNKS)
        def _():
            pltpu.sync_copy(acc.at[pl.ds(rc * RCHUNK, RCHUNK)],
                            rows0.at[pl.ds(0, RCHUNK)])
            pltpu.sync_copy(rows0.at[pl.ds(0, RCHUNK)],
                            out_hbm.at[c, pl.ds(rc * RCHUNK, RCHUNK)])


_SC_AGG_CACHE = []


def _sc_agg_fn():
    if not _SC_AGG_CACHE:
        _SC_AGG_CACHE.append(pl.kernel(
            _sc_body,
            out_type=jax.ShapeDtypeStruct((NC, N_NODES, DIM), jnp.float32),
            mesh=plsc.VectorSubcoreMesh(core_axis_name="c",
                                        subcore_axis_name="s",
                                        num_cores=NC, num_subcores=NS),
            scratch_types=[
                pltpu.VMEM_SHARED((N_NODES, DIM), jnp.float32),
                pltpu.VMEM((G, CHUNK), jnp.int32),
                pltpu.VMEM((G, CHUNK), jnp.int32),
                pltpu.VMEM((G, CHUNK), jnp.float32),
                pltpu.VMEM((CHUNK, DIM), jnp.float32),
                pltpu.VMEM((CHUNK, DIM), jnp.float32),
                pltpu.SemaphoreType.DMA,
                pltpu.SemaphoreType.DMA,
                pltpu.SemaphoreType.DMA,
            ],
        ))
    return _SC_AGG_CACHE[0]


BM = 1000


def _mm_body(x_ref, a1_ref, a2_ref, wln_ref, w1_ref, w2_ref,
             o0_ref, o1_ref, o2_ref):
    o0_ref[...] = jnp.dot(x_ref[...], wln_ref[...],
                          preferred_element_type=jnp.float32)
    o1_ref[...] = jnp.dot(a1_ref[0], w1_ref[...],
                          preferred_element_type=jnp.float32)
    o2_ref[...] = jnp.dot(a2_ref[0], w2_ref[...],
                          preferred_element_type=jnp.float32)


_row_spec = pl.BlockSpec((BM, DIM), lambda i: (i, 0))
_w_spec = pl.BlockSpec((DIM, DIM), lambda i: (0, 0))

_tc_matmul = pl.pallas_call(
    _mm_body,
    grid=(N_NODES // BM,),
    in_specs=[_row_spec,
              pl.BlockSpec((1, BM, DIM), lambda i: (0, i, 0)),
              pl.BlockSpec((1, BM, DIM), lambda i: (1, i, 0)),
              _w_spec, _w_spec, _w_spec],
    out_specs=[_row_spec, _row_spec, _row_spec],
    out_shape=[jax.ShapeDtypeStruct((N_NODES, DIM), jnp.float32)] * 3,
)


def kernel(x, edge_index, edge_attr, edge_index2, edge_attr2, W_ln, W1, W2):
    ei1 = edge_index.astype(jnp.int32)
    ei2 = edge_index2.astype(jnp.int32)
    npad = N_EDGES_PAD - N_EDGES
    pad_idx = (jnp.arange(npad, dtype=jnp.int32) * 8) % N_NODES
    pad_attr = jnp.zeros((npad,), jnp.float32)

    def _pad(a, p):
        return jnp.concatenate([a, p])

    src = jnp.stack([_pad(ei1[0], pad_idx), _pad(ei2[0], pad_idx)])
    dst = jnp.stack([_pad(ei1[1], pad_idx), _pad(ei2[1], pad_idx)])
    attr = jnp.stack([_pad(edge_attr, pad_attr), _pad(edge_attr2, pad_attr)])
    src = src.reshape(NC, N_CHUNKS_PAD, CHUNK)
    dst = dst.reshape(NC, N_CHUNKS_PAD, CHUNK)
    attr = attr.reshape(NC, N_CHUNKS_PAD, CHUNK)
    agg = _sc_agg_fn()(src, dst, attr, x)
    x0, x1, x2 = _tc_matmul(x, agg, agg, W_ln, W1, W2)
    return (x0, x1, x2)
